# baseline (speedup 1.0000x reference)
import functools

import jax
import jax.numpy as jnp
from jax import lax
from jax.experimental import pallas as pl
from jax.experimental.pallas import tpu as pltpu
from jax.experimental.pallas import tpu_sc as plsc

D = 128
NC = 2
NS = 16
NW = NC * NS
NB = 6


def _make_gather(batch, seq):
    assert batch % NW == 0
    bpw = batch // NW
    nchunks = seq
    full_groups = nchunks // NB
    rem = nchunks % NB

    mesh = plsc.VectorSubcoreMesh(core_axis_name="c", subcore_axis_name="s")

    @functools.partial(
        pl.kernel,
        mesh=mesh,
        out_type=jax.ShapeDtypeStruct((seq, batch, D), jnp.float32),
        scratch_types=(
            [pltpu.VMEM((seq, bpw), jnp.int32)]
            + [pltpu.VMEM((bpw, D), jnp.float32)] * NB
            + [pltpu.SemaphoreType.DMA] * (2 * NB)
        ),
    )
    def gather_kernel(idx_hbm, table_hbm, out_hbm, idx_v, *bufs_sems):
        bufs = bufs_sems[:NB]
        gs = bufs_sems[NB:2 * NB]
        ws = bufs_sems[2 * NB:]
        wid = lax.axis_index("s") * NC + lax.axis_index("c")
        col_base = wid * bpw

        def gather_start(c, buf, sem):
            pltpu.async_copy(table_hbm.at[idx_v.at[c]], buf, sem)

        def gather_wait(buf, sem):
            pltpu.make_async_copy(
                table_hbm.at[idx_v.at[0]], buf, sem).wait()

        def write_start(c, buf, sem):
            pltpu.async_copy(buf, out_hbm.at[c, pl.ds(col_base, bpw)], sem)

        def write_wait(buf, sem):
            pltpu.make_async_copy(
                buf, out_hbm.at[0, pl.ds(col_base, bpw)], sem).wait()

        assert NB - 1 <= 8 < seq
        pltpu.sync_copy(idx_hbm.at[pl.ds(0, 8), pl.ds(col_base, bpw)],
                        idx_v.at[pl.ds(0, 8)])

        for k in range(NB - 1):
            gather_start(k, bufs[k], gs[k])
        pltpu.sync_copy(idx_hbm.at[pl.ds(8, seq - 8), pl.ds(col_base, bpw)],
                        idx_v.at[pl.ds(8, seq - 8)])

        def body(i, carry):
            for k in range(NB):
                c = NB * i + k
                kk = (k + NB - 1) % NB

                def refill(c=c, kk=kk):
                    write_wait(bufs[kk], ws[kk])
                    gather_start(c + NB - 1, bufs[kk], gs[kk])

                if k == 0:
                    pl.when(i > 0)(refill)
                    pl.when(i == 0)(
                        lambda: gather_start(NB - 1, bufs[NB - 1], gs[NB - 1]))
                else:
                    pl.when(c + NB - 1 < nchunks)(refill)
                gather_wait(bufs[k], gs[k])
                write_start(c, bufs[k], ws[k])
            return carry

        lax.fori_loop(0, full_groups, body, 0)
        for k in range(rem):
            c = NB * full_groups + k
            gather_wait(bufs[k], gs[k])
            write_start(c, bufs[k], ws[k])
        for k in range(NB):
            write_wait(bufs[k], ws[k])

    return gather_kernel


def kernel(input, weight):
    batch, seq = input.shape
    idx_t = input.astype(jnp.int32).T
    out_t = _make_gather(batch, seq)(idx_t, weight)
    return jnp.transpose(out_t, (1, 0, 2))


# speedup vs baseline: 1.0106x; 1.0106x over previous
import functools

import jax
import jax.numpy as jnp
from jax import lax
from jax.experimental import pallas as pl
from jax.experimental.pallas import tpu as pltpu
from jax.experimental.pallas import tpu_sc as plsc

D = 128
NC = 2
NS = 16
NW = NC * NS
NB = 6


def _make_gather(batch, seq):
    assert batch % NW == 0
    bpw = batch // NW
    nchunks = seq
    full_groups = nchunks // NB
    rem = nchunks % NB

    mesh = plsc.VectorSubcoreMesh(core_axis_name="c", subcore_axis_name="s")

    @functools.partial(
        pl.kernel,
        mesh=mesh,
        out_type=jax.ShapeDtypeStruct((seq, batch, D), jnp.float32),
        scratch_types=(
            [pltpu.VMEM((seq, bpw), jnp.int32)]
            + [pltpu.VMEM((bpw, D), jnp.float32)] * NB
            + [pltpu.SemaphoreType.DMA] * (2 * NB)
        ),
    )
    def gather_kernel(idx_hbm, table_hbm, out_hbm, idx_v, *bufs_sems):
        bufs = bufs_sems[:NB]
        gs = bufs_sems[NB:2 * NB]
        ws = bufs_sems[2 * NB:]
        wid = lax.axis_index("s") * NC + lax.axis_index("c")
        col_base = wid * bpw

        def gather_start(c, buf, sem):
            pltpu.async_copy(table_hbm.at[idx_v.at[c]], buf, sem)

        def gather_wait(buf, sem):
            pltpu.make_async_copy(
                table_hbm.at[idx_v.at[0]], buf, sem).wait()

        def write_start(c, buf, sem):
            pltpu.async_copy(buf, out_hbm.at[c, pl.ds(col_base, bpw)], sem)

        def write_wait(buf, sem):
            pltpu.make_async_copy(
                buf, out_hbm.at[0, pl.ds(col_base, bpw)], sem).wait()

        assert NB - 1 <= 8 < seq
        pltpu.sync_copy(idx_hbm.at[pl.ds(0, 8), pl.ds(col_base, bpw)],
                        idx_v.at[pl.ds(0, 8)])

        for k in range(NB - 1):
            gather_start(k, bufs[k], gs[k])
        pltpu.sync_copy(idx_hbm.at[pl.ds(8, seq - 8), pl.ds(col_base, bpw)],
                        idx_v.at[pl.ds(8, seq - 8)])

        def body(i, carry):
            for k in range(NB):
                c = NB * i + k
                gather_wait(bufs[k], gs[k])
                write_start(c, bufs[k], ws[k])
                kk = (k + NB - 1) % NB

                def refill(c=c, kk=kk):
                    write_wait(bufs[kk], ws[kk])
                    gather_start(c + NB - 1, bufs[kk], gs[kk])

                if k == 0:
                    pl.when(i > 0)(refill)
                    pl.when(i == 0)(
                        lambda: gather_start(NB - 1, bufs[NB - 1], gs[NB - 1]))
                else:
                    pl.when(c + NB - 1 < nchunks)(refill)
            return carry

        lax.fori_loop(0, full_groups, body, 0)
        for k in range(rem):
            c = NB * full_groups + k
            gather_wait(bufs[k], gs[k])
            write_start(c, bufs[k], ws[k])
        for k in range(NB):
            write_wait(bufs[k], ws[k])

    return gather_kernel


def kernel(input, weight):
    batch, seq = input.shape
    idx_t = input.astype(jnp.int32).T
    out_t = _make_gather(batch, seq)(idx_t, weight)
    return jnp.transpose(out_t, (1, 0, 2))
